# baseline (speedup 1.0000x reference)
import functools

import jax
import jax.numpy as jnp
from jax import lax
from jax.experimental import pallas as pl
from jax.experimental.pallas import tpu as pltpu
from jax.experimental.pallas import tpu_sc as plsc

N = 100000
D = 512
L = 16
NBLK = D // L

_C0 = 0.99836373
_C1 = -0.31610295
_C2 = 0.098738074
_C3 = -0.022229603
_C4 = 0.0033113218
_C5 = -0.00031363618
_C6 = 1.8048671e-05
_C7 = -5.734537e-07
_C8 = 7.700704e-09


def _sc_counts():
    try:
        info = plsc.get_sparse_core_info()
        return info.num_cores, info.num_subcores
    except Exception:
        return 2, 16


def _bus_kernel(f0_hbm, f1_hbm, bus_hbm, table_hbm, out_hbm,
                f0_v, f1_v, bus_v, table_v, outbuf_v,
                *, nc, ns, rows_w, chunk):
    wid = lax.axis_index("s") * nc + lax.axis_index("c")
    pltpu.sync_copy(f0_hbm.at[wid], f0_v.at[pl.ds(0, rows_w)])
    pltpu.sync_copy(f1_hbm.at[wid], f1_v.at[pl.ds(0, rows_w)])
    pltpu.sync_copy(bus_hbm.at[wid], bus_v.at[pl.ds(0, rows_w)])
    pltpu.sync_copy(table_hbm, table_v)

    nchunks = rows_w // chunk
    base_row = wid * rows_w

    def chunk_body(k, _):
        @plsc.parallel_loop(0, chunk, unroll=2)
        def row_body(r):
            i = k * chunk + r
            t = bus_v[pl.ds(i, L)][0]
            f0 = f0_v[pl.ds(i, L)][0]
            f1 = f1_v[pl.ds(i, L)][0]
            for j in range(NBLK):
                sl = pl.ds(j * L, L)
                w0 = table_v[t, 0, sl]
                w1 = table_v[t, 1, sl]
                bb = table_v[t, 2, sl]
                x = f0 * w0 + f1 * w1 + bb
                xc = jnp.minimum(jnp.maximum(x, -4.0), 4.0)
                u = xc * xc
                u2 = u * u
                u4 = u2 * u2
                e0 = _C0 + _C1 * u
                e1 = _C2 + _C3 * u
                e2 = _C4 + _C5 * u
                e3 = _C6 + _C7 * u
                h0 = e0 + e1 * u2
                h1 = e2 + e3 * u2
                g = h0 + (h1 + _C8 * u4) * u4
                outbuf_v[r, sl] = xc * g
        pltpu.sync_copy(outbuf_v, out_hbm.at[pl.ds(base_row + k * chunk, chunk)])
        return 0

    lax.fori_loop(0, nchunks, chunk_body, 0)


def kernel(feat, bus_type, W_slack, b_slack, W_gen, b_gen, W_load, b_load):
    nc, ns = _sc_counts()
    nw = nc * ns
    rows_w = N // nw
    chunk = 125

    z = jnp.zeros((3, D), jnp.float32)
    mk = lambda W, b: jnp.concatenate([W, b[None, :]], axis=0)
    table = jnp.stack([z, mk(W_slack, b_slack), mk(W_gen, b_gen),
                       mk(W_load, b_load)])

    f0s = feat[:, 0].reshape(nw, rows_w)
    f1s = feat[:, 1].reshape(nw, rows_w)
    bus3 = bus_type.astype(jnp.int32).reshape(nw, rows_w)

    mesh = plsc.VectorSubcoreMesh(core_axis_name="c", subcore_axis_name="s",
                                  num_cores=nc, num_subcores=ns)
    run = pl.kernel(
        functools.partial(_bus_kernel, nc=nc, ns=ns, rows_w=rows_w,
                          chunk=chunk),
        out_type=jax.ShapeDtypeStruct((N, D), jnp.float32),
        mesh=mesh,
        compiler_params=pltpu.CompilerParams(use_tc_tiling_on_sc=False),
        scratch_types=[
            pltpu.VMEM((rows_w + L,), jnp.float32),
            pltpu.VMEM((rows_w + L,), jnp.float32),
            pltpu.VMEM((rows_w + L,), jnp.int32),
            pltpu.VMEM((4, 3, D), jnp.float32),
            pltpu.VMEM((chunk, D), jnp.float32),
        ],
    )
    return run(f0s, f1s, bus3, table)


# speedup vs baseline: 1.4514x; 1.4514x over previous
import functools

import jax
import jax.numpy as jnp
from jax import lax
from jax.experimental import pallas as pl
from jax.experimental.pallas import tpu as pltpu
from jax.experimental.pallas import tpu_sc as plsc

N = 100000
D = 512
L = 16
NBLK = D // L

_C0 = 0.99836373
_C1 = -0.31610295
_C2 = 0.098738074
_C3 = -0.022229603
_C4 = 0.0033113218
_C5 = -0.00031363618
_C6 = 1.8048671e-05
_C7 = -5.734537e-07
_C8 = 7.700704e-09


def _sc_counts():
    try:
        info = plsc.get_sparse_core_info()
        return info.num_cores, info.num_subcores
    except Exception:
        return 2, 16


def _bus_kernel(f0_hbm, f1_hbm, bus_hbm, table_hbm, out_hbm,
                f0_v, f1_v, bus_v, table_v, outbuf_v,
                *, nc, ns, rows_w, chunk):
    wid = lax.axis_index("s") * nc + lax.axis_index("c")
    pltpu.sync_copy(f0_hbm.at[wid], f0_v.at[pl.ds(0, rows_w)])
    pltpu.sync_copy(f1_hbm.at[wid], f1_v.at[pl.ds(0, rows_w)])
    pltpu.sync_copy(bus_hbm.at[wid], bus_v.at[pl.ds(0, rows_w)])
    pltpu.sync_copy(table_hbm, table_v)

    nchunks = rows_w // chunk
    base_row = wid * rows_w

    def chunk_body(k, _):
        def row_body(r, _):
            i = k * chunk + r
            t = bus_v[pl.ds(i, L)][0]
            f0 = f0_v[pl.ds(i, L)][0]
            f1 = f1_v[pl.ds(i, L)][0]

            @plsc.parallel_loop(0, NBLK, unroll=4)
            def blk_body(j):
                sl = pl.ds(j * L, L)
                w0 = table_v[t, 0, sl]
                w1 = table_v[t, 1, sl]
                bb = table_v[t, 2, sl]
                x = f0 * w0 + f1 * w1 + bb
                xc = jnp.minimum(jnp.maximum(x, -4.0), 4.0)
                u = xc * xc
                u2 = u * u
                u4 = u2 * u2
                e0 = _C0 + _C1 * u
                e1 = _C2 + _C3 * u
                e2 = _C4 + _C5 * u
                e3 = _C6 + _C7 * u
                h0 = e0 + e1 * u2
                h1 = e2 + e3 * u2
                g = h0 + (h1 + _C8 * u4) * u4
                outbuf_v[r, sl] = xc * g
            return 0

        lax.fori_loop(0, chunk, row_body, 0)
        pltpu.sync_copy(outbuf_v, out_hbm.at[pl.ds(base_row + k * chunk, chunk)])
        return 0

    lax.fori_loop(0, nchunks, chunk_body, 0)


def kernel(feat, bus_type, W_slack, b_slack, W_gen, b_gen, W_load, b_load):
    nc, ns = _sc_counts()
    nw = nc * ns
    rows_w = N // nw
    chunk = 125

    z = jnp.zeros((3, D), jnp.float32)
    mk = lambda W, b: jnp.concatenate([W, b[None, :]], axis=0)
    table = jnp.stack([z, mk(W_slack, b_slack), mk(W_gen, b_gen),
                       mk(W_load, b_load)])

    f0s = feat[:, 0].reshape(nw, rows_w)
    f1s = feat[:, 1].reshape(nw, rows_w)
    bus3 = bus_type.astype(jnp.int32).reshape(nw, rows_w)

    mesh = plsc.VectorSubcoreMesh(core_axis_name="c", subcore_axis_name="s",
                                  num_cores=nc, num_subcores=ns)
    run = pl.kernel(
        functools.partial(_bus_kernel, nc=nc, ns=ns, rows_w=rows_w,
                          chunk=chunk),
        out_type=jax.ShapeDtypeStruct((N, D), jnp.float32),
        mesh=mesh,
        compiler_params=pltpu.CompilerParams(use_tc_tiling_on_sc=False),
        scratch_types=[
            pltpu.VMEM((rows_w + L,), jnp.float32),
            pltpu.VMEM((rows_w + L,), jnp.float32),
            pltpu.VMEM((rows_w + L,), jnp.int32),
            pltpu.VMEM((4, 3, D), jnp.float32),
            pltpu.VMEM((chunk, D), jnp.float32),
        ],
    )
    return run(f0s, f1s, bus3, table)


# speedup vs baseline: 2.0248x; 1.3951x over previous
import functools

import jax
import jax.numpy as jnp
from jax import lax
from jax.experimental import pallas as pl
from jax.experimental.pallas import tpu as pltpu
from jax.experimental.pallas import tpu_sc as plsc

N = 100000
D = 512
L = 16
NBLK = D // L

_C0 = 0.99836373
_C1 = -0.31610295
_C2 = 0.098738074
_C3 = -0.022229603
_C4 = 0.0033113218
_C5 = -0.00031363618
_C6 = 1.8048671e-05
_C7 = -5.734537e-07
_C8 = 7.700704e-09


def _sc_counts():
    try:
        info = plsc.get_sparse_core_info()
        return info.num_cores, info.num_subcores
    except Exception:
        return 2, 16


def _bus_kernel(f0_hbm, f1_hbm, bus_hbm, table_hbm, out_hbm,
                f0_v, f1_v, bus_v, table_v, outbuf_v,
                *, nc, ns, rows_w, chunk):
    wid = lax.axis_index("s") * nc + lax.axis_index("c")
    pltpu.sync_copy(f0_hbm.at[wid], f0_v.at[pl.ds(0, rows_w)])
    pltpu.sync_copy(f1_hbm.at[wid], f1_v.at[pl.ds(0, rows_w)])
    pltpu.sync_copy(bus_hbm.at[wid], bus_v.at[pl.ds(0, rows_w)])
    pltpu.sync_copy(table_hbm, table_v)

    nchunks = rows_w // chunk
    base_row = wid * rows_w

    def chunk_body(k, _):
        def row_body(r, _):
            i = k * chunk + r
            t = bus_v[pl.ds(i, L)][0]
            f0 = f0_v[pl.ds(i, L)][0]
            f1 = f1_v[pl.ds(i, L)][0]

            @plsc.parallel_loop(0, NBLK, unroll=4)
            def blk_body(j):
                sl = pl.ds(j * L, L)
                w0 = table_v[t, 0, sl]
                w1 = table_v[t, 1, sl]
                bb = table_v[t, 2, sl]
                x = f0 * w0 + f1 * w1 + bb
                e = jnp.exp(x + x)
                outbuf_v[r, sl] = 1.0 - 2.0 / (e + 1.0)
            return 0

        lax.fori_loop(0, chunk, row_body, 0)
        pltpu.sync_copy(outbuf_v, out_hbm.at[pl.ds(base_row + k * chunk, chunk)])
        return 0

    lax.fori_loop(0, nchunks, chunk_body, 0)


def kernel(feat, bus_type, W_slack, b_slack, W_gen, b_gen, W_load, b_load):
    nc, ns = _sc_counts()
    nw = nc * ns
    rows_w = N // nw
    chunk = 125

    z = jnp.zeros((3, D), jnp.float32)
    mk = lambda W, b: jnp.concatenate([W, b[None, :]], axis=0)
    table = jnp.stack([z, mk(W_slack, b_slack), mk(W_gen, b_gen),
                       mk(W_load, b_load)])

    f0s = feat[:, 0].reshape(nw, rows_w)
    f1s = feat[:, 1].reshape(nw, rows_w)
    bus3 = bus_type.astype(jnp.int32).reshape(nw, rows_w)

    mesh = plsc.VectorSubcoreMesh(core_axis_name="c", subcore_axis_name="s",
                                  num_cores=nc, num_subcores=ns)
    run = pl.kernel(
        functools.partial(_bus_kernel, nc=nc, ns=ns, rows_w=rows_w,
                          chunk=chunk),
        out_type=jax.ShapeDtypeStruct((N, D), jnp.float32),
        mesh=mesh,
        compiler_params=pltpu.CompilerParams(use_tc_tiling_on_sc=False),
        scratch_types=[
            pltpu.VMEM((rows_w + L,), jnp.float32),
            pltpu.VMEM((rows_w + L,), jnp.float32),
            pltpu.VMEM((rows_w + L,), jnp.int32),
            pltpu.VMEM((4, 3, D), jnp.float32),
            pltpu.VMEM((chunk, D), jnp.float32),
        ],
    )
    return run(f0s, f1s, bus3, table)


# speedup vs baseline: 2.1898x; 1.0815x over previous
import functools

import jax
import jax.numpy as jnp
from jax import lax
from jax.experimental import pallas as pl
from jax.experimental.pallas import tpu as pltpu
from jax.experimental.pallas import tpu_sc as plsc

N = 100000
D = 512
L = 16
NBLK = D // L
BPI = 2
LOG2_JPI = 4


def _sc_counts():
    try:
        info = plsc.get_sparse_core_info()
        return info.num_cores, info.num_subcores
    except Exception:
        return 2, 16


def _bus_kernel(bus_hbm, pf_hbm, table_hbm, out_hbm, bus_v, pf_v, table_v,
                outbuf_v, sem, *, nc, ns, rows_w, chunk):
    wid = lax.axis_index("s") * nc + lax.axis_index("c")
    pltpu.sync_copy(bus_hbm.at[wid], bus_v.at[pl.ds(0, rows_w)])
    pltpu.sync_copy(pf_hbm.at[wid], pf_v.at[pl.ds(0, 2 * rows_w)])
    pltpu.sync_copy(table_hbm, table_v)

    nchunks = rows_w // chunk
    base_row = wid * rows_w
    jpi = NBLK // BPI

    def chunk_body(k, _):
        buf = lax.rem(k, 2)

        @pl.when(k >= 2)
        def _():
            pltpu.make_async_copy(
                out_hbm.at[pl.ds(0, chunk)], outbuf_v.at[0], sem).wait()

        @plsc.parallel_loop(0, chunk * jpi, unroll=4)
        def q_body(q):
            r = lax.shift_right_logical(q, LOG2_JPI)
            jq = lax.bitwise_and(q, jpi - 1)
            i = k * chunk + r
            t = bus_v[pl.ds(i, L)][0]
            fv = pf_v[pl.ds(2 * i, L)]
            f0 = fv[0]
            f1 = fv[1]
            base = t * (3 * D)
            for s in range(BPI):
                col = (jq * BPI + s) * L
                w0 = table_v[pl.ds(base + col, L)]
                w1 = table_v[pl.ds(base + col + D, L)]
                bb = table_v[pl.ds(base + col + 2 * D, L)]
                x = f0 * w0 + f1 * w1 + bb
                e = jnp.exp(x + x)
                outbuf_v[buf, r, pl.ds(col, L)] = 1.0 - 2.0 / (e + 1.0)

        pltpu.async_copy(
            outbuf_v.at[buf],
            out_hbm.at[pl.ds(base_row + k * chunk, chunk)], sem)
        return 0

    lax.fori_loop(0, nchunks, chunk_body, 0)

    for _ in range(2):
        pltpu.make_async_copy(
            out_hbm.at[pl.ds(0, chunk)], outbuf_v.at[0], sem).wait()


def kernel(feat, bus_type, W_slack, b_slack, W_gen, b_gen, W_load, b_load):
    nc, ns = _sc_counts()
    nw = nc * ns
    rows_w = N // nw
    chunk = 25

    z = jnp.zeros((3, D), jnp.float32)
    mk = lambda W, b: jnp.concatenate([W, b[None, :]], axis=0)
    table = jnp.stack([z, mk(W_slack, b_slack), mk(W_gen, b_gen),
                       mk(W_load, b_load)]).reshape(-1)

    bus3 = bus_type.astype(jnp.int32).reshape(nw, rows_w)
    pf = feat.reshape(nw, 2 * rows_w)

    mesh = plsc.VectorSubcoreMesh(core_axis_name="c", subcore_axis_name="s",
                                  num_cores=nc, num_subcores=ns)
    run = pl.kernel(
        functools.partial(_bus_kernel, nc=nc, ns=ns, rows_w=rows_w,
                          chunk=chunk),
        out_type=jax.ShapeDtypeStruct((N, D), jnp.float32),
        mesh=mesh,
        compiler_params=pltpu.CompilerParams(use_tc_tiling_on_sc=False),
        scratch_types=[
            pltpu.VMEM((rows_w + L,), jnp.int32),
            pltpu.VMEM((2 * rows_w + L,), jnp.float32),
            pltpu.VMEM((4 * 3 * D,), jnp.float32),
            pltpu.VMEM((2, chunk, D), jnp.float32),
            pltpu.SemaphoreType.DMA,
        ],
    )
    return run(bus3, pf, table)
